# baseline (speedup 1.0000x reference)
import jax
import jax.numpy as jnp
from jax import lax
from jax.experimental import pallas as pl
from jax.experimental.pallas import tpu as pltpu
from jax.experimental.pallas import tpu_sc as plsc

LANES = 16
LATENT = 32
NUM_WORKERS = 32
B_PER_W = 512
IDX_ROWS = 4
IDX_COLS = 128


_GATHER_DN = lax.GatherDimensionNumbers(
    offset_dims=(), collapsed_slice_dims=(0,), start_index_map=(0,))


def _shuffle(vec, idx):
    return lax.gather(vec, idx[:, None], _GATHER_DN, slice_sizes=(1,),
                      mode=lax.GatherScatterMode.PROMISE_IN_BOUNDS)


def _sc_body(uid_hbm, iid_hbm, eu_hbm, ei_hbm, out_hbm,
             idx_u, idx_i, rows_u, rows_i, out_v, sem):
    wid = lax.axis_index("s") * 2 + lax.axis_index("c")

    pltpu.sync_copy(uid_hbm.at[pl.ds(wid * IDX_ROWS, IDX_ROWS)], idx_u)
    pltpu.sync_copy(iid_hbm.at[pl.ds(wid * IDX_ROWS, IDX_ROWS)], idx_i)

    copies = []
    for c in range(IDX_ROWS):
        dst = pl.ds(c * IDX_COLS, IDX_COLS)
        copies.append(pltpu.async_copy(eu_hbm.at[idx_u.at[c]], rows_u.at[dst], sem))
        copies.append(pltpu.async_copy(ei_hbm.at[idx_i.at[c]], rows_i.at[dst], sem))
    for cp in copies:
        cp.wait()

    iota = lax.iota(jnp.int32, LANES)
    shuf = [iota ^ k for k in (8, 4, 2, 1)]

    def group(g, carry):
        base = g * LANES
        acc = jnp.zeros((LANES,), jnp.float32)
        for j in range(LANES):
            b = base + j
            p = (rows_u[b, pl.ds(0, LANES)] * rows_i[b, pl.ds(0, LANES)]
                 + rows_u[b, pl.ds(LANES, LANES)] * rows_i[b, pl.ds(LANES, LANES)])
            for s in shuf:
                p = p + _shuffle(p, s)
            acc = jnp.where(iota == j, p, acc)
        out_v[pl.ds(base, LANES)] = acc
        return carry

    lax.fori_loop(0, B_PER_W // LANES, group, 0)

    pltpu.sync_copy(out_v, out_hbm.at[pl.ds(wid * B_PER_W, B_PER_W)])


def kernel(user_id, item_id, emb_user, emb_item):
    batch = user_id.shape[0]
    uid2 = user_id.reshape(NUM_WORKERS * IDX_ROWS, IDX_COLS).astype(jnp.int32)
    iid2 = item_id.reshape(NUM_WORKERS * IDX_ROWS, IDX_COLS).astype(jnp.int32)

    mesh = plsc.VectorSubcoreMesh(core_axis_name="c", subcore_axis_name="s")
    run = pl.kernel(
        _sc_body,
        out_type=jax.ShapeDtypeStruct((batch,), jnp.float32),
        mesh=mesh,
        compiler_params=pltpu.CompilerParams(use_tc_tiling_on_sc=False),
        scratch_types=[
            pltpu.VMEM((IDX_ROWS, IDX_COLS), jnp.int32),
            pltpu.VMEM((IDX_ROWS, IDX_COLS), jnp.int32),
            pltpu.VMEM((B_PER_W, LATENT), jnp.float32),
            pltpu.VMEM((B_PER_W, LATENT), jnp.float32),
            pltpu.VMEM((B_PER_W,), jnp.float32),
            pltpu.SemaphoreType.DMA,
        ],
    )
    out = run(uid2, iid2, emb_user, emb_item)
    return out.reshape(batch, 1)
